# baseline (speedup 1.0000x reference)
import jax
import jax.numpy as jnp
from jax.experimental import pallas as pl
from jax.experimental.pallas import tpu as pltpu
from math import ceil

W, H, L = 1280, 720, 16
NBW = int(ceil(W / L))
NBH = int(ceil(H / L))
NUM_BLOCK = NBW * NBH
N_POINTS = 32768

TB = 160
GRID = ceil(NUM_BLOCK / TB)


def _screen_kernel(x_ref, y_ref, r_ref, o_ref):
    x = x_ref[...]
    y = y_ref[...]
    r = r_ref[...]
    xmin = jnp.clip(x - r, 0, W).astype(jnp.int32)
    ymin = jnp.clip(y - r, 0, H).astype(jnp.int32)
    xmax = jnp.clip(x + r, 0, W).astype(jnp.int32)
    ymax = jnp.clip(y + r, 0, H).astype(jnp.int32)

    t = pl.program_id(0) * TB + jax.lax.broadcasted_iota(jnp.int32, (TB, 1), 0)
    xi = t // NBH
    yi = t - xi * NBH
    left = xi * L
    top = yi * L
    right = jnp.minimum(left + L, W)
    bottom = jnp.minimum(top + L, H)

    overlap_x = jnp.minimum(xmax, right) > jnp.maximum(xmin, left)
    overlap_y = jnp.minimum(ymax, bottom) > jnp.maximum(ymin, top)
    o_ref[...] = overlap_x & overlap_y


def kernel(pos2d, radius):
    x = pos2d[:, 0].reshape(1, N_POINTS)
    y = pos2d[:, 1].reshape(1, N_POINTS)
    r = radius.reshape(1, N_POINTS)
    row_spec = pl.BlockSpec((1, N_POINTS), lambda i: (0, 0))
    return pl.pallas_call(
        _screen_kernel,
        out_shape=jax.ShapeDtypeStruct((NUM_BLOCK, N_POINTS), jnp.bool_),
        grid=(GRID,),
        in_specs=[row_spec, row_spec, row_spec],
        out_specs=pl.BlockSpec((TB, N_POINTS), lambda i: (i, 0)),
        compiler_params=pltpu.CompilerParams(
            dimension_semantics=("parallel",),
        ),
        name="screen_tile_mask",
    )(x, y, r)


# speedup vs baseline: 1.0860x; 1.0860x over previous
import jax
import jax.numpy as jnp
from jax.experimental import pallas as pl
from jax.experimental.pallas import tpu as pltpu
import numpy as np
from math import ceil

W, H, L = 1280, 720, 16
NBW = int(ceil(W / L))
NBH = int(ceil(H / L))
NUM_BLOCK = NBW * NBH
N_POINTS = 32768

XI_PER_STEP = 8
ROWS = XI_PER_STEP * NBH
BN = 8192

def _screen_kernel(x_ref, y_ref, r_ref, o_ref, oyt_ref):
    i = pl.program_id(1)
    x = x_ref[...]
    y = y_ref[...]
    r = r_ref[...]
    xmin = jnp.clip(x - r, 0, W).astype(jnp.int32)
    xmax = jnp.clip(x + r, 0, W).astype(jnp.int32)

    rows = jax.lax.broadcasted_iota(jnp.int32, (ROWS, 1), 0)
    xi_loc = rows // NBH
    top = (rows - xi_loc * NBH) * L

    @pl.when(i == 0)
    def _():
        ymin = jnp.clip(y - r, 0, H).astype(jnp.int32)
        ymax = jnp.clip(y + r, 0, H).astype(jnp.int32)
        nonempty = (xmax > xmin) & (ymax > ymin)
        oyt_ref[...] = ((ymax > top) & (ymin < top + L)) & nonempty

    left = (xi_loc + i * XI_PER_STEP) * L
    ox = (xmax > left) & (xmin < left + L)
    o_ref[...] = oyt_ref[...] & ox


def kernel(pos2d, radius):
    x = pos2d[:, 0].reshape(1, N_POINTS)
    y = pos2d[:, 1].reshape(1, N_POINTS)
    r = radius.reshape(1, N_POINTS)
    row_spec = pl.BlockSpec((1, BN), lambda j, i: (0, j))
    return pl.pallas_call(
        _screen_kernel,
        out_shape=jax.ShapeDtypeStruct((NUM_BLOCK, N_POINTS), jnp.bool_),
        grid=(N_POINTS // BN, NBW // XI_PER_STEP),
        in_specs=[row_spec, row_spec, row_spec],
        out_specs=pl.BlockSpec((ROWS, BN), lambda j, i: (i, j)),
        scratch_shapes=[pltpu.VMEM((ROWS, BN), jnp.bool_)],
        compiler_params=pltpu.CompilerParams(
            dimension_semantics=("arbitrary", "arbitrary"),
        ),
        name="screen_tile_mask",
    )(x, y, r)


# speedup vs baseline: 2.1269x; 1.9584x over previous
import jax
import jax.numpy as jnp
from jax.experimental import pallas as pl
from jax.experimental.pallas import tpu as pltpu
from math import ceil

W, H, L = 1280, 720, 16
NBW = int(ceil(W / L))
NBH = int(ceil(H / L))
NUM_BLOCK = NBW * NBH
N_POINTS = 32768

XI_PER_STEP = 8
ROWS = XI_PER_STEP * NBH
BN = 8192


def _screen_kernel(x_ref, y_ref, r_ref, o_ref, oy_ref):
    i = pl.program_id(1)
    x = x_ref[...]
    y = y_ref[...]
    r = r_ref[...]
    xmin = jnp.clip(x - r, 0, W).astype(jnp.int32)
    xmax = jnp.clip(x + r, 0, W).astype(jnp.int32)

    @pl.when(i == 0)
    def _():
        ymin = jnp.clip(y - r, 0, H).astype(jnp.int32)
        ymax = jnp.clip(y + r, 0, H).astype(jnp.int32)
        top = jax.lax.broadcasted_iota(jnp.int32, (NBH, 1), 0) * L
        oy = (ymax > top) & (ymin < top + L) & (xmax > xmin) & (ymax > ymin)
        oy_ref[...] = oy.astype(jnp.int8)

    oy = oy_ref[...]
    for k in range(XI_PER_STEP):
        left = (i * XI_PER_STEP + k) * L
        ox = ((xmax > left) & (xmin < left + L)).astype(jnp.int8)
        o_ref[k * NBH:(k + 1) * NBH, :] = oy & ox


def kernel(pos2d, radius):
    x = pos2d[:, 0].reshape(1, N_POINTS)
    y = pos2d[:, 1].reshape(1, N_POINTS)
    r = radius.reshape(1, N_POINTS)
    row_spec = pl.BlockSpec((1, BN), lambda j, i: (0, j))
    out = pl.pallas_call(
        _screen_kernel,
        out_shape=jax.ShapeDtypeStruct((NUM_BLOCK, N_POINTS), jnp.int8),
        grid=(N_POINTS // BN, NBW // XI_PER_STEP),
        in_specs=[row_spec, row_spec, row_spec],
        out_specs=pl.BlockSpec((ROWS, BN), lambda j, i: (i, j)),
        scratch_shapes=[pltpu.VMEM((NBH, BN), jnp.int8)],
        compiler_params=pltpu.CompilerParams(
            dimension_semantics=("arbitrary", "arbitrary"),
        ),
        name="screen_tile_mask",
    )(x, y, r)
    return out.astype(jnp.bool_)


# speedup vs baseline: 2.1273x; 1.0002x over previous
import jax
import jax.numpy as jnp
from jax.experimental import pallas as pl
from jax.experimental.pallas import tpu as pltpu
from math import ceil

W, H, L = 1280, 720, 16
NBW = int(ceil(W / L))
NBH = int(ceil(H / L))
NUM_BLOCK = NBW * NBH
N_POINTS = 32768

XI_PER_STEP = 8
ROWS = XI_PER_STEP * NBH
BN = 8192


def _screen_kernel(x_ref, y_ref, r_ref, o_ref, oy_ref):
    i = pl.program_id(1)
    x = x_ref[...]
    y = y_ref[...]
    r = r_ref[...]
    xmin = jnp.clip(x - r, 0, W).astype(jnp.int32)
    xmax = jnp.clip(x + r, 0, W).astype(jnp.int32)

    @pl.when(i == 0)
    def _():
        ymin = jnp.clip(y - r, 0, H).astype(jnp.int32)
        ymax = jnp.clip(y + r, 0, H).astype(jnp.int32)
        top = jax.lax.broadcasted_iota(jnp.int32, (NBH, 1), 0) * L
        oy = (ymax > top) & (ymin < top + L) & (xmax > xmin) & (ymax > ymin)
        oy_ref[...] = oy.astype(jnp.int8)

    oy = oy_ref[...]
    for k in range(XI_PER_STEP):
        left = (i * XI_PER_STEP + k) * L
        ox = ((xmax > left) & (xmin < left + L)).astype(jnp.int8)
        o_ref[k * NBH:(k + 1) * NBH, :] = oy & ox


def kernel(pos2d, radius):
    x = pos2d[:, 0].reshape(1, N_POINTS)
    y = pos2d[:, 1].reshape(1, N_POINTS)
    r = radius.reshape(1, N_POINTS)
    row_spec = pl.BlockSpec((1, BN), lambda j, i: (0, j))
    out = pl.pallas_call(
        _screen_kernel,
        out_shape=jax.ShapeDtypeStruct((NUM_BLOCK, N_POINTS), jnp.int8),
        grid=(N_POINTS // BN, NBW // XI_PER_STEP),
        in_specs=[row_spec, row_spec, row_spec],
        out_specs=pl.BlockSpec((ROWS, BN), lambda j, i: (i, j)),
        scratch_shapes=[pltpu.VMEM((NBH, BN), jnp.int8)],
        compiler_params=pltpu.CompilerParams(
            dimension_semantics=("parallel", "arbitrary"),
        ),
        name="screen_tile_mask",
    )(x, y, r)
    return out.astype(jnp.bool_)
